# Initial kernel scaffold; baseline (speedup 1.0000x reference)
import jax
import jax.numpy as jnp
from jax.experimental import pallas as pl


def kernel(x, luts):
    raise NotImplementedError("write your pallas kernel here")


# speedup vs baseline: 132.3445x; 132.3445x over previous
import functools

import jax
import jax.numpy as jnp
from jax import lax
from jax.experimental import pallas as pl
from jax.experimental.pallas import tpu as pltpu
from jax.experimental.pallas import tpu_sc as plsc

NUM_INPUTS = 4096
ADDRESS_SIZE = 4
NUM_LUTS = NUM_INPUTS // ADDRESS_SIZE
ENTRY_PER_LUT = 2 ** ADDRESS_SIZE
BATCH = 4096

_INFO = plsc.get_sparse_core_info()
_NC = _INFO.num_cores
_NS = _INFO.num_subcores
_L = _INFO.num_lanes
_NW = _NC * _NS
_ROWS_PER_W = BATCH // _NW
_G = _L
_NGROUPS = _ROWS_PER_W // _G


def _make_kernel():
    mesh = plsc.VectorSubcoreMesh(core_axis_name="c", subcore_axis_name="s")

    @functools.partial(
        pl.kernel,
        mesh=mesh,
        compiler_params=pltpu.CompilerParams(needs_layout_passes=False),
        out_type=jax.ShapeDtypeStruct((BATCH,), jnp.float32),
        scratch_types=[
            pltpu.VMEM((NUM_LUTS * ENTRY_PER_LUT,), jnp.float32),
            pltpu.VMEM((_G * NUM_INPUTS,), jnp.int32),
            pltpu.VMEM((_G,), jnp.float32),
        ],
    )
    def k(x_hbm, luts_hbm, out_hbm, luts_v, xg_v, out_v):
        wid = lax.axis_index("s") * _NC + lax.axis_index("c")
        pltpu.sync_copy(luts_hbm, luts_v)
        lane = lax.broadcasted_iota(jnp.int32, (_L,), 0)
        row_off = lane * NUM_INPUTS

        for g in range(_NGROUPS):
            row0 = wid * _ROWS_PER_W + g * _G
            pltpu.sync_copy(
                x_hbm.at[pl.ds(row0 * NUM_INPUTS, _G * NUM_INPUTS)], xg_v)

            def body(l, acc):
                col = row_off + jnp.broadcast_to(l * ADDRESS_SIZE, (_L,))
                b0 = plsc.load_gather(xg_v, [col])
                b1 = plsc.load_gather(xg_v, [col + 1])
                b2 = plsc.load_gather(xg_v, [col + 2])
                b3 = plsc.load_gather(xg_v, [col + 3])
                addr = b0 + (b1 << 1) + (b2 << 2) + (b3 << 3)
                lut_idx = addr + jnp.broadcast_to(l * ENTRY_PER_LUT, (_L,))
                val = plsc.load_gather(luts_v, [lut_idx])
                return acc + val

            acc = lax.fori_loop(0, NUM_LUTS, body,
                                jnp.zeros((_L,), jnp.float32))
            out_v[...] = acc
            pltpu.sync_copy(out_v, out_hbm.at[pl.ds(row0, _G)])

    return k


_kernel_call = _make_kernel()


@jax.jit
def kernel(x, luts):
    x32 = x.astype(jnp.int32)
    out = _kernel_call(x32.reshape(-1), luts.reshape(-1))
    return out.reshape(BATCH, 1)
